# Initial kernel scaffold; baseline (speedup 1.0000x reference)
import jax
import jax.numpy as jnp
from jax.experimental import pallas as pl


def kernel(batch_embedding, batch_target):
    raise NotImplementedError("write your pallas kernel here")


# speedup vs baseline: 9.5129x; 9.5129x over previous
import jax
import jax.numpy as jnp
from jax.experimental import pallas as pl
from jax.experimental.pallas import tpu as pltpu

DELTA_VAR = 0.5
DELTA_D = 2.5
ALPHA = 1.0
BETA = 1.0
GAMMA = 0.001

K = 16


def _loss_body(e_ref, seg_ref, out_ref):
    E = e_ref[:, :]
    seg = seg_ref[:, :]
    D, N = E.shape

    kio = jax.lax.broadcasted_iota(jnp.int32, (K, N), 0)
    P = (kio == seg).astype(jnp.float32)

    counts = jnp.sum(P, axis=1, keepdims=True)
    present = counts > 0.0
    C = jnp.sum(present.astype(jnp.float32))
    safe_counts = jnp.where(present, counts, 1.0)

    sums = jax.lax.dot_general(
        E, P, (((1,), (1,)), ((), ())),
        preferred_element_type=jnp.float32)
    inv_counts = (1.0 / safe_counts)
    mu = sums * inv_counts.reshape(1, K)

    xnorm2 = jnp.sum(E * E, axis=0, keepdims=True)
    S = jax.lax.dot_general(
        mu, E, (((0,), (0,)), ((), ())),
        preferred_element_type=jnp.float32)
    munorm2 = jnp.sum(mu * mu, axis=0, keepdims=True)
    s_sel = jnp.sum(P * S, axis=0, keepdims=True)
    mn_sel = jax.lax.dot_general(
        munorm2, P, (((1,), (0,)), ((), ())),
        preferred_element_type=jnp.float32)
    d2 = jnp.maximum(xnorm2 - 2.0 * s_sel + mn_sel, 0.0)
    d_pix = jnp.sqrt(d2 + 1e-12)
    hv = jnp.maximum(d_pix - DELTA_VAR, 0.0) ** 2

    per_seg = jax.lax.dot_general(
        P, hv, (((1,), (1,)), ((), ())),
        preferred_element_type=jnp.float32)
    per_seg = per_seg * inv_counts
    var_term = jnp.sum(jnp.where(present, per_seg, 0.0)) / C

    rows = []
    for a in range(K):
        da = mu - mu[:, a:a + 1]
        rows.append(jnp.sum(da * da, axis=0, keepdims=True))
    dist2 = jnp.concatenate(rows, axis=0)
    dist = jnp.sqrt(dist2 + 1e-8)
    hinge_d = jnp.maximum(2.0 * DELTA_D - dist, 0.0) ** 2
    pr = present.astype(jnp.float32)
    pair = pr * pr.reshape(1, K)
    ia = jax.lax.broadcasted_iota(jnp.int32, (K, K), 0)
    ib = jax.lax.broadcasted_iota(jnp.int32, (K, K), 1)
    mask = jnp.where(ia == ib, 0.0, pair)
    denom = jnp.maximum(C * (C - 1.0), 1.0)
    dist_term = jnp.where(C > 1.0, jnp.sum(hinge_d * mask) / denom,
                          jnp.float32(0.0))

    norms = jnp.sqrt(munorm2 + 1e-12)
    reg_term = jnp.sum(jnp.where(present.reshape(1, K), norms, 0.0)) / C

    out_ref[0, 0] = ALPHA * var_term + BETA * dist_term + GAMMA * reg_term


def kernel(batch_embedding, batch_target):
    D = batch_embedding.shape[1]
    N = batch_embedding.shape[2] * batch_embedding.shape[3]
    E = batch_embedding[0].reshape(D, N)
    seg = batch_target[0].reshape(1, N)
    loss = pl.pallas_call(
        _loss_body,
        out_shape=jax.ShapeDtypeStruct((1, 1), jnp.float32),
        out_specs=pl.BlockSpec(memory_space=pltpu.SMEM),
    )(E, seg)
    return loss[0, 0]
